# baseline (speedup 1.0000x reference)
import functools

import jax
import jax.numpy as jnp
from jax.experimental import pallas as pl
from jax.experimental.pallas import tpu as pltpu

NUM_PARTS = 8
POSE_NUM = 16
NC = 48
G = 512
PD = 20
H = 128
MAT_MODE = ((0, 1), (0, 2), (1, 2))


def _pose_select_body(ap_ref, qp_ref, pid_ref, wscale_ref):
    ap = ap_ref[...]
    qp = qp_ref[...]

    def quat(aa):
        angle = jnp.sqrt(jnp.sum(aa * aa, axis=-1))
        half = 0.5 * angle
        small = angle < 1e-6
        sinc = jnp.where(small, 0.5 - angle * angle / 48.0,
                         jnp.sin(half) / jnp.where(small, 1.0, angle))
        return jnp.cos(half), aa * sinc[..., None]

    kw, kxyz = quat(ap)
    qw, qxyz = quat(qp)
    dot = kw * qw + jnp.sum(kxyz * qxyz, axis=-1)
    absdot = jnp.abs(dot)
    iota = jax.lax.broadcasted_iota(jnp.int32, (POSE_NUM, 1), 0)
    for p in range(NUM_PARTS):
        pd = jnp.sum(absdot[:, 3 * p:3 * p + 3], axis=-1, keepdims=True)
        maxv = jnp.max(pd)
        pid = jnp.min(jnp.where(pd == maxv, iota, POSE_NUM))
        pid_ref[p] = pid.astype(jnp.int32)
        wscale_ref[p] = maxv / jnp.maximum(maxv, 1e-16)


def _main_body(pid_ref, wscale_ref, tpts_ref, view_ref, tflag_ref, dists_ref,
               tb_ref, tcat_ref, feat_ref, w1_ref, b1_ref, w2_ref, b2_ref,
               raw_ref, occ_ref, occs_ref):
    B = tpts_ref.shape[0]
    xyz_min = tb_ref[0, 0, :]
    xyz_max = tb_ref[0, 1, :]
    scale = 2.0 / (xyz_max - xyz_min)
    dists = dists_ref[...]
    lane_iota = jax.lax.broadcasted_iota(jnp.int32, (B, G), 1)

    acc_raw = jnp.zeros((B, 4), jnp.float32)
    acc_occ = jnp.zeros((B, 1), jnp.float32)
    occ_cols = []
    for p in range(NUM_PARTS):
        pid = pid_ref[p]
        wscale = wscale_ref[p]
        pts = tpts_ref[:, p, :]
        pn = (pts - xyz_min) * scale - 1.0
        g = jnp.clip((pn + 1.0) * (0.5 * (G - 1)), 0.0, G - 1.0)
        g0f = jnp.floor(g)
        t = g - g0f
        g0 = g0f.astype(jnp.int32)
        lines = []
        for ax in range(3):
            onehot = (lane_iota == g0[:, ax:ax + 1]).astype(jnp.bfloat16)
            tab = tcat_ref[ax, pid]
            vv = jnp.dot(onehot, tab, preferred_element_type=jnp.float32)
            v0 = vv[:, :NC]
            v1 = vv[:, NC:]
            lines.append(v0 + t[:, ax:ax + 1] * (v1 - v0))
        planes = []
        for ip, (i0, i1) in enumerate(MAT_MODE):
            prod = lines[i0] * lines[i1]
            fl = feat_ref[ip, pid]
            planes.append(jnp.dot(prod, fl, preferred_element_type=jnp.float32))
        feat = jnp.concatenate(planes, axis=-1) * wscale
        inp = jnp.concatenate([pts, view_ref[:, p, :], feat], axis=-1)
        h = jnp.maximum(
            jnp.dot(inp, w1_ref[p], preferred_element_type=jnp.float32) + b1_ref[p, :],
            0.0)
        out = jnp.dot(h, w2_ref[p], preferred_element_type=jnp.float32) + b2_ref[p, :]
        m = tflag_ref[:, p:p + 1]
        raw_p = out[:, :4] * m
        occ_p = (1.0 - jnp.exp(-jnp.maximum(out[:, 4:5], 0.0) * dists)) * m
        acc_raw = acc_raw + raw_p
        acc_occ = acc_occ + occ_p
        occ_cols.append(occ_p)
    raw_ref[...] = acc_raw * (1.0 / NUM_PARTS)
    occ_ref[...] = acc_occ * (1.0 / NUM_PARTS)
    occs_ref[...] = jnp.concatenate(occ_cols, axis=-1)


@jax.jit
def kernel(tpts, viewdir, tflag, dists, part_dist, poses, all_poses, tbounds,
           coord_line, feat_line, W1, b1, W2, b2):
    del part_dist
    N = tpts.shape[0]

    ap3 = all_poses.reshape(POSE_NUM, 24, 3)
    qp3 = poses.reshape(1, 24, 3)
    pid, wscale = pl.pallas_call(
        _pose_select_body,
        out_shape=(
            jax.ShapeDtypeStruct((NUM_PARTS,), jnp.int32),
            jax.ShapeDtypeStruct((NUM_PARTS,), jnp.float32),
        ),
        out_specs=(
            pl.BlockSpec(memory_space=pltpu.SMEM),
            pl.BlockSpec(memory_space=pltpu.SMEM),
        ),
    )(ap3, qp3)

    tt = jnp.swapaxes(coord_line, -1, -2)
    tt_shift = jnp.concatenate([tt[:, :, 1:, :], tt[:, :, -1:, :]], axis=2)
    tcat = jnp.concatenate([tt, tt_shift], axis=-1).astype(jnp.bfloat16)

    tflag_f = tflag.astype(jnp.float32)
    dists2 = dists.reshape(N, 1)

    B = 512
    grid = (N // B,)
    out_shapes = (
        jax.ShapeDtypeStruct((N, 4), jnp.float32),
        jax.ShapeDtypeStruct((N, 1), jnp.float32),
        jax.ShapeDtypeStruct((N, NUM_PARTS), jnp.float32),
    )
    raw, occ, occs = pl.pallas_call(
        _main_body,
        grid=grid,
        in_specs=[
            pl.BlockSpec(memory_space=pltpu.SMEM),
            pl.BlockSpec(memory_space=pltpu.SMEM),
            pl.BlockSpec((B, NUM_PARTS, 3), lambda i: (i, 0, 0)),
            pl.BlockSpec((B, NUM_PARTS, 3), lambda i: (i, 0, 0)),
            pl.BlockSpec((B, NUM_PARTS), lambda i: (i, 0)),
            pl.BlockSpec((B, 1), lambda i: (i, 0)),
            pl.BlockSpec((1, 2, 3), lambda i: (0, 0, 0)),
            pl.BlockSpec((3, POSE_NUM, G, 2 * NC), lambda i: (0, 0, 0, 0)),
            pl.BlockSpec((3, POSE_NUM, NC, PD), lambda i: (0, 0, 0, 0)),
            pl.BlockSpec((NUM_PARTS, 66, H), lambda i: (0, 0, 0)),
            pl.BlockSpec((NUM_PARTS, H), lambda i: (0, 0)),
            pl.BlockSpec((NUM_PARTS, H, 5), lambda i: (0, 0, 0)),
            pl.BlockSpec((NUM_PARTS, 5), lambda i: (0, 0)),
        ],
        out_specs=(
            pl.BlockSpec((B, 4), lambda i: (i, 0)),
            pl.BlockSpec((B, 1), lambda i: (i, 0)),
            pl.BlockSpec((B, NUM_PARTS), lambda i: (i, 0)),
        ),
        out_shape=out_shapes,
    )(pid, wscale, tpts, viewdir, tflag_f, dists2, tbounds, tcat, feat_line,
      W1, b1, W2, b2)
    return raw, occ, occs.reshape(N, NUM_PARTS, 1)


# speedup vs baseline: 1.7772x; 1.7772x over previous
import jax
import jax.numpy as jnp
from jax.experimental import pallas as pl
from jax.experimental.pallas import tpu as pltpu

NUM_PARTS = 8
POSE_NUM = 16
NC = 48
G = 512
PD = 20
H = 128
MAT_MODE = ((0, 1), (0, 2), (1, 2))


def _pose_select_body(ap_ref, qp_ref, pid_ref, wscale_ref):
    ap = ap_ref[...]
    qp = qp_ref[...]

    def quat(aa):
        angle = jnp.sqrt(jnp.sum(aa * aa, axis=-1))
        half = 0.5 * angle
        small = angle < 1e-6
        sinc = jnp.where(small, 0.5 - angle * angle / 48.0,
                         jnp.sin(half) / jnp.where(small, 1.0, angle))
        return jnp.cos(half), aa * sinc[..., None]

    kw, kxyz = quat(ap)
    qw, qxyz = quat(qp)
    dot = kw * qw + jnp.sum(kxyz * qxyz, axis=-1)
    absdot = jnp.abs(dot)
    iota = jax.lax.broadcasted_iota(jnp.int32, (POSE_NUM, 1), 0)
    for p in range(NUM_PARTS):
        pd = jnp.sum(absdot[:, 3 * p:3 * p + 3], axis=-1, keepdims=True)
        maxv = jnp.max(pd)
        pid = jnp.min(jnp.where(pd == maxv, iota, POSE_NUM))
        pid_ref[p] = pid.astype(jnp.int32)
        wscale_ref[p] = maxv / jnp.maximum(maxv, 1e-16)


def _main_body(pid_ref, tpts_ref, view_ref, tflag_ref, dists_ref, tb_ref,
               tab_ref, feat_ref, w1_ref, b1_ref, w2_ref, b2_ref, wsc_ref,
               raw_ref, occ_ref, occs_ref):
    p = pl.program_id(1)
    B = dists_ref.shape[0]
    xyz_min = tb_ref[0, 0, :]
    xyz_max = tb_ref[0, 1, :]
    scale = 2.0 / (xyz_max - xyz_min)
    dists = dists_ref[...]
    lane_iota = jax.lax.broadcasted_iota(jnp.int32, (B, G), 1)
    iota8 = jax.lax.broadcasted_iota(jnp.int32, (B, NUM_PARTS), 1)

    pts = tpts_ref[0]
    pn = (pts - xyz_min) * scale - 1.0
    g = jnp.clip((pn + 1.0) * (0.5 * (G - 1)), 0.0, G - 1.0)
    g0f = jnp.floor(g)
    t = g - g0f
    g0 = g0f.astype(jnp.int32)
    lines = []
    for ax in range(3):
        onehot = (lane_iota == g0[:, ax:ax + 1]).astype(jnp.bfloat16)
        vv = jnp.dot(onehot, tab_ref[ax, 0], preferred_element_type=jnp.float32)
        v0 = vv[:, :NC]
        v1 = vv[:, NC:]
        lines.append(v0 + t[:, ax:ax + 1] * (v1 - v0))
    planes = []
    for ip, (i0, i1) in enumerate(MAT_MODE):
        prod = lines[i0] * lines[i1]
        planes.append(jnp.dot(prod, feat_ref[ip, 0],
                              preferred_element_type=jnp.float32))
    feat = jnp.concatenate(planes, axis=-1) * wsc_ref[p]
    inp = jnp.concatenate([pts, view_ref[0], feat], axis=-1)
    h = jnp.maximum(
        jnp.dot(inp, w1_ref[0], preferred_element_type=jnp.float32) + b1_ref[0, 0, :],
        0.0)
    out = jnp.dot(h, w2_ref[0], preferred_element_type=jnp.float32) + b2_ref[0, 0, :]
    m = jnp.sum(tflag_ref[...] * (iota8 == p).astype(jnp.float32),
                axis=1, keepdims=True)
    raw_p = out[:, :4] * m
    occ_p = (1.0 - jnp.exp(-jnp.maximum(out[:, 4:5], 0.0) * dists)) * m

    @pl.when(p == 0)
    def _():
        raw_ref[...] = jnp.zeros_like(raw_ref)
        occ_ref[...] = jnp.zeros_like(occ_ref)
        occs_ref[...] = jnp.zeros_like(occs_ref)

    raw_ref[...] += raw_p * (1.0 / NUM_PARTS)
    occ_ref[...] += occ_p * (1.0 / NUM_PARTS)
    occs_ref[...] += occ_p * (iota8 == p).astype(jnp.float32)


@jax.jit
def kernel(tpts, viewdir, tflag, dists, part_dist, poses, all_poses, tbounds,
           coord_line, feat_line, W1, b1, W2, b2):
    del part_dist
    N = tpts.shape[0]

    ap3 = all_poses.reshape(POSE_NUM, 24, 3)
    qp3 = poses.reshape(1, 24, 3)
    pid, wscale = pl.pallas_call(
        _pose_select_body,
        out_shape=(
            jax.ShapeDtypeStruct((NUM_PARTS,), jnp.int32),
            jax.ShapeDtypeStruct((NUM_PARTS,), jnp.float32),
        ),
        out_specs=(
            pl.BlockSpec(memory_space=pltpu.SMEM),
            pl.BlockSpec(memory_space=pltpu.SMEM),
        ),
    )(ap3, qp3)

    tt = jnp.swapaxes(coord_line, -1, -2)
    tt_shift = jnp.concatenate([tt[:, :, 1:, :], tt[:, :, -1:, :]], axis=2)
    tcat = jnp.concatenate([tt, tt_shift], axis=-1).astype(jnp.bfloat16)

    tpts_p = jnp.swapaxes(tpts, 0, 1)
    view_p = jnp.swapaxes(viewdir, 0, 1)
    tflag_f = tflag.astype(jnp.float32)
    dists2 = dists.reshape(N, 1)

    B = 2048
    grid = (N // B, NUM_PARTS)
    out_shapes = (
        jax.ShapeDtypeStruct((N, 4), jnp.float32),
        jax.ShapeDtypeStruct((N, 1), jnp.float32),
        jax.ShapeDtypeStruct((N, NUM_PARTS), jnp.float32),
    )
    grid_spec = pltpu.PrefetchScalarGridSpec(
        num_scalar_prefetch=1,
        grid=grid,
        in_specs=[
            pl.BlockSpec((1, B, 3), lambda i, p, pid_ref: (p, i, 0)),
            pl.BlockSpec((1, B, 3), lambda i, p, pid_ref: (p, i, 0)),
            pl.BlockSpec((B, NUM_PARTS), lambda i, p, pid_ref: (i, 0)),
            pl.BlockSpec((B, 1), lambda i, p, pid_ref: (i, 0)),
            pl.BlockSpec((1, 2, 3), lambda i, p, pid_ref: (0, 0, 0)),
            pl.BlockSpec((3, 1, G, 2 * NC),
                         lambda i, p, pid_ref: (0, pid_ref[p], 0, 0)),
            pl.BlockSpec((3, 1, NC, PD),
                         lambda i, p, pid_ref: (0, pid_ref[p], 0, 0)),
            pl.BlockSpec((1, 66, H), lambda i, p, pid_ref: (p, 0, 0)),
            pl.BlockSpec((1, 1, H), lambda i, p, pid_ref: (p, 0, 0)),
            pl.BlockSpec((1, H, 5), lambda i, p, pid_ref: (p, 0, 0)),
            pl.BlockSpec((1, 1, 5), lambda i, p, pid_ref: (p, 0, 0)),
            pl.BlockSpec(memory_space=pltpu.SMEM),
        ],
        out_specs=(
            pl.BlockSpec((B, 4), lambda i, p, pid_ref: (i, 0)),
            pl.BlockSpec((B, 1), lambda i, p, pid_ref: (i, 0)),
            pl.BlockSpec((B, NUM_PARTS), lambda i, p, pid_ref: (i, 0)),
        ),
    )
    raw, occ, occs = pl.pallas_call(
        _main_body,
        grid_spec=grid_spec,
        out_shape=out_shapes,
    )(pid, tpts_p, view_p, tflag_f, dists2, tbounds, tcat, feat_line,
      W1, b1.reshape(NUM_PARTS, 1, H), W2, b2.reshape(NUM_PARTS, 1, 5), wscale)
    return raw, occ, occs.reshape(N, NUM_PARTS, 1)


# speedup vs baseline: 1.7913x; 1.0079x over previous
import jax
import jax.numpy as jnp
from jax.experimental import pallas as pl
from jax.experimental.pallas import tpu as pltpu

NUM_PARTS = 8
POSE_NUM = 16
NC = 48
G = 512
PD = 20
H = 128
MAT_MODE = ((0, 1), (0, 2), (1, 2))


def _pose_select_body(ap_ref, qp_ref, pid_ref, wscale_ref):
    ap = ap_ref[...]
    qp = qp_ref[...]

    def quat(aa):
        angle = jnp.sqrt(jnp.sum(aa * aa, axis=-1))
        half = 0.5 * angle
        small = angle < 1e-6
        sinc = jnp.where(small, 0.5 - angle * angle / 48.0,
                         jnp.sin(half) / jnp.where(small, 1.0, angle))
        return jnp.cos(half), aa * sinc[..., None]

    kw, kxyz = quat(ap)
    qw, qxyz = quat(qp)
    dot = kw * qw + jnp.sum(kxyz * qxyz, axis=-1)
    absdot = jnp.abs(dot)
    iota = jax.lax.broadcasted_iota(jnp.int32, (POSE_NUM, 1), 0)
    for p in range(NUM_PARTS):
        pd = jnp.sum(absdot[:, 3 * p:3 * p + 3], axis=-1, keepdims=True)
        maxv = jnp.max(pd)
        pid = jnp.min(jnp.where(pd == maxv, iota, POSE_NUM))
        pid_ref[p] = pid.astype(jnp.int32)
        wscale_ref[p] = maxv / jnp.maximum(maxv, 1e-16)


def _main_body(pid_ref, tpts_ref, pv_ref, tflag_ref, dists_ref, tb_ref,
               tab_ref, feat_ref, w1pv_ref, w1f_ref, b1_ref, w2_ref, b2_ref,
               wsc_ref, raw_ref, occ_ref, occs_ref):
    p = pl.program_id(1)
    B = dists_ref.shape[0]
    xyz_min = tb_ref[0, 0, :]
    xyz_max = tb_ref[0, 1, :]
    scale = 2.0 / (xyz_max - xyz_min)
    dists = dists_ref[...]
    lane_iota = jax.lax.broadcasted_iota(jnp.int32, (B, G), 1)
    lane96 = jax.lax.broadcasted_iota(jnp.int32, (B, 2 * NC), 1)
    iota8 = jax.lax.broadcasted_iota(jnp.int32, (B, NUM_PARTS), 1)
    r96 = jax.lax.broadcasted_iota(jnp.int32, (2 * NC, NC), 0)
    c48 = jax.lax.broadcasted_iota(jnp.int32, (2 * NC, NC), 1)
    fold = jnp.logical_or(r96 == c48, r96 == c48 + NC).astype(jnp.bfloat16)

    pts = tpts_ref[0]
    pn = (pts - xyz_min) * scale - 1.0
    g = jnp.clip((pn + 1.0) * (0.5 * (G - 1)), 0.0, G - 1.0)
    g0f = jnp.floor(g)
    t = g - g0f
    g0 = g0f.astype(jnp.int32)
    lines = []
    for ax in range(3):
        onehot = (lane_iota == g0[:, ax:ax + 1]).astype(jnp.bfloat16)
        vv = jnp.dot(onehot, tab_ref[ax, 0], preferred_element_type=jnp.float32)
        tax = t[:, ax:ax + 1]
        ww = jnp.where(lane96 < NC, 1.0 - tax, tax)
        uu = (vv * ww).astype(jnp.bfloat16)
        lines.append(jnp.dot(uu, fold, preferred_element_type=jnp.float32))
    h_pre = jnp.dot(pv_ref[0], w1pv_ref[0], preferred_element_type=jnp.float32)
    wsc = wsc_ref[p]
    for ip, (i0, i1) in enumerate(MAT_MODE):
        prod = lines[i0] * lines[i1]
        fw = jnp.dot(feat_ref[ip, 0], w1f_ref[0, ip],
                     preferred_element_type=jnp.float32) * wsc
        h_pre = h_pre + jnp.dot(prod, fw, preferred_element_type=jnp.float32)
    h = jnp.maximum(h_pre + b1_ref[0, 0, :], 0.0)
    out = jnp.dot(h, w2_ref[0], preferred_element_type=jnp.float32) + b2_ref[0, 0, :]
    m = jnp.sum(tflag_ref[...] * (iota8 == p).astype(jnp.float32),
                axis=1, keepdims=True)
    raw_p = out[:, :4] * m
    occ_p = (1.0 - jnp.exp(-jnp.maximum(out[:, 4:5], 0.0) * dists)) * m

    @pl.when(p == 0)
    def _():
        raw_ref[...] = jnp.zeros_like(raw_ref)
        occ_ref[...] = jnp.zeros_like(occ_ref)
        occs_ref[...] = jnp.zeros_like(occs_ref)

    raw_ref[...] += raw_p * (1.0 / NUM_PARTS)
    occ_ref[...] += occ_p * (1.0 / NUM_PARTS)
    occs_ref[...] += occ_p * (iota8 == p).astype(jnp.float32)


@jax.jit
def kernel(tpts, viewdir, tflag, dists, part_dist, poses, all_poses, tbounds,
           coord_line, feat_line, W1, b1, W2, b2):
    del part_dist
    N = tpts.shape[0]

    ap3 = all_poses.reshape(POSE_NUM, 24, 3)
    qp3 = poses.reshape(1, 24, 3)
    pid, wscale = pl.pallas_call(
        _pose_select_body,
        out_shape=(
            jax.ShapeDtypeStruct((NUM_PARTS,), jnp.int32),
            jax.ShapeDtypeStruct((NUM_PARTS,), jnp.float32),
        ),
        out_specs=(
            pl.BlockSpec(memory_space=pltpu.SMEM),
            pl.BlockSpec(memory_space=pltpu.SMEM),
        ),
    )(ap3, qp3)

    tt = jnp.swapaxes(coord_line, -1, -2)
    tt_shift = jnp.concatenate([tt[:, :, 1:, :], tt[:, :, -1:, :]], axis=2)
    tcat = jnp.concatenate([tt, tt_shift], axis=-1).astype(jnp.bfloat16)

    tpts_p = jnp.swapaxes(tpts, 0, 1)
    view_p = jnp.swapaxes(viewdir, 0, 1)
    pv = jnp.concatenate([tpts_p, view_p], axis=-1)
    tflag_f = tflag.astype(jnp.float32)
    dists2 = dists.reshape(N, 1)
    W1pv = W1[:, :6, :]
    W1f = W1[:, 6:, :].reshape(NUM_PARTS, 3, PD, H)

    B = 2048
    grid = (N // B, NUM_PARTS)
    out_shapes = (
        jax.ShapeDtypeStruct((N, 4), jnp.float32),
        jax.ShapeDtypeStruct((N, 1), jnp.float32),
        jax.ShapeDtypeStruct((N, NUM_PARTS), jnp.float32),
    )
    grid_spec = pltpu.PrefetchScalarGridSpec(
        num_scalar_prefetch=1,
        grid=grid,
        in_specs=[
            pl.BlockSpec((1, B, 3), lambda i, p, pid_ref: (p, i, 0)),
            pl.BlockSpec((1, B, 6), lambda i, p, pid_ref: (p, i, 0)),
            pl.BlockSpec((B, NUM_PARTS), lambda i, p, pid_ref: (i, 0)),
            pl.BlockSpec((B, 1), lambda i, p, pid_ref: (i, 0)),
            pl.BlockSpec((1, 2, 3), lambda i, p, pid_ref: (0, 0, 0)),
            pl.BlockSpec((3, 1, G, 2 * NC),
                         lambda i, p, pid_ref: (0, pid_ref[p], 0, 0)),
            pl.BlockSpec((3, 1, NC, PD),
                         lambda i, p, pid_ref: (0, pid_ref[p], 0, 0)),
            pl.BlockSpec((1, 6, H), lambda i, p, pid_ref: (p, 0, 0)),
            pl.BlockSpec((1, 3, PD, H), lambda i, p, pid_ref: (p, 0, 0, 0)),
            pl.BlockSpec((1, 1, H), lambda i, p, pid_ref: (p, 0, 0)),
            pl.BlockSpec((1, H, 5), lambda i, p, pid_ref: (p, 0, 0)),
            pl.BlockSpec((1, 1, 5), lambda i, p, pid_ref: (p, 0, 0)),
            pl.BlockSpec(memory_space=pltpu.SMEM),
        ],
        out_specs=(
            pl.BlockSpec((B, 4), lambda i, p, pid_ref: (i, 0)),
            pl.BlockSpec((B, 1), lambda i, p, pid_ref: (i, 0)),
            pl.BlockSpec((B, NUM_PARTS), lambda i, p, pid_ref: (i, 0)),
        ),
    )
    raw, occ, occs = pl.pallas_call(
        _main_body,
        grid_spec=grid_spec,
        out_shape=out_shapes,
    )(pid, tpts_p, pv, tflag_f, dists2, tbounds, tcat, feat_line,
      W1pv, W1f, b1.reshape(NUM_PARTS, 1, H), W2, b2.reshape(NUM_PARTS, 1, 5),
      wscale)
    return raw, occ, occs.reshape(N, NUM_PARTS, 1)


# speedup vs baseline: 4.3824x; 2.4465x over previous
import jax
import jax.numpy as jnp
from jax.experimental import pallas as pl
from jax.experimental.pallas import tpu as pltpu

NUM_PARTS = 8
POSE_NUM = 16
NC = 48
G = 512
PD = 20
H = 128
MAT_MODE = ((0, 1), (0, 2), (1, 2))


def _pose_select_body(ap_ref, qp_ref, pid_ref, wscale_ref):
    ap = ap_ref[...]
    qp = qp_ref[...]

    def quat(aa):
        angle = jnp.sqrt(jnp.sum(aa * aa, axis=-1))
        half = 0.5 * angle
        small = angle < 1e-6
        sinc = jnp.where(small, 0.5 - angle * angle / 48.0,
                         jnp.sin(half) / jnp.where(small, 1.0, angle))
        return jnp.cos(half), aa * sinc[..., None]

    kw, kxyz = quat(ap)
    qw, qxyz = quat(qp)
    dot = kw * qw + jnp.sum(kxyz * qxyz, axis=-1)
    absdot = jnp.abs(dot)
    iota = jax.lax.broadcasted_iota(jnp.int32, (POSE_NUM, 1), 0)
    for p in range(NUM_PARTS):
        pd = jnp.sum(absdot[:, 3 * p:3 * p + 3], axis=-1, keepdims=True)
        maxv = jnp.max(pd)
        pid = jnp.min(jnp.where(pd == maxv, iota, POSE_NUM))
        pid_ref[p] = pid.astype(jnp.int32)
        wscale_ref[p] = maxv / jnp.maximum(maxv, 1e-16)


def _main_body(pid_ref, tpts_ref, view_ref, tflag_ref, dists_ref, tb_ref,
               tab_ref, feat_ref, w1p_ref, w1v_ref, w1f_ref, b1_ref, w2_ref,
               b2_ref, wsc_ref, raw_ref, occ_ref, occs_ref):
    p = pl.program_id(1)
    B = dists_ref.shape[0]
    dists = dists_ref[...]
    lane_iota = jax.lax.broadcasted_iota(jnp.int32, (B, G), 1)
    lane96 = jax.lax.broadcasted_iota(jnp.int32, (B, 2 * NC), 1)
    lane24 = jax.lax.broadcasted_iota(jnp.int32, (B, 24), 1)
    iota8 = jax.lax.broadcasted_iota(jnp.int32, (B, NUM_PARTS), 1)
    sub24 = jax.lax.broadcasted_iota(jnp.int32, (24, 1), 0)
    r96 = jax.lax.broadcasted_iota(jnp.int32, (2 * NC, NC), 0)
    c48 = jax.lax.broadcasted_iota(jnp.int32, (2 * NC, NC), 1)
    fold = jnp.logical_or(r96 == c48, r96 == c48 + NC).astype(jnp.bfloat16)

    tpts24 = tpts_ref[...]
    view24 = view_ref[...]
    lines = []
    for ax in range(3):
        sel = (lane24 == 3 * p + ax).astype(jnp.float32)
        x = jnp.sum(tpts24 * sel, axis=1, keepdims=True)
        xmin = tb_ref[0, 0, ax]
        xmax = tb_ref[0, 1, ax]
        gax = jnp.clip((x - xmin) * ((G - 1.0) / (xmax - xmin)), 0.0, G - 1.0)
        g0f = jnp.floor(gax)
        tax = gax - g0f
        g0 = g0f.astype(jnp.int32)
        onehot = (lane_iota == g0).astype(jnp.bfloat16)
        vv = jax.lax.dot_general(
            onehot, tab_ref[ax, 0], (((1,), (1,)), ((), ())),
            preferred_element_type=jnp.float32)
        ww = jnp.where(lane96 < NC, 1.0 - tax, tax)
        uu = (vv * ww).astype(jnp.bfloat16)
        lines.append(jnp.dot(uu, fold, preferred_element_type=jnp.float32))

    pmask = jnp.logical_and(sub24 >= 3 * p, sub24 < 3 * p + 3)
    pm = pmask.astype(jnp.float32)
    h_pre = (jnp.dot(tpts24, w1p_ref[...] * pm,
                     preferred_element_type=jnp.float32)
             + jnp.dot(view24, w1v_ref[...] * pm,
                       preferred_element_type=jnp.float32))
    wsc = wsc_ref[p]
    for ip, (i0, i1) in enumerate(MAT_MODE):
        prod = lines[i0] * lines[i1]
        fw = jnp.dot(feat_ref[ip, 0], w1f_ref[0, ip],
                     preferred_element_type=jnp.float32) * wsc
        h_pre = h_pre + jnp.dot(prod, fw, preferred_element_type=jnp.float32)
    h = jnp.maximum(h_pre + b1_ref[0, 0, :], 0.0)
    out = jnp.dot(h, w2_ref[0], preferred_element_type=jnp.float32) + b2_ref[0, 0, :]
    m = jnp.sum(tflag_ref[...] * (iota8 == p).astype(jnp.float32),
                axis=1, keepdims=True)
    raw_p = out[:, :4] * m
    occ_p = (1.0 - jnp.exp(-jnp.maximum(out[:, 4:5], 0.0) * dists)) * m

    @pl.when(p == 0)
    def _():
        raw_ref[...] = jnp.zeros_like(raw_ref)
        occ_ref[...] = jnp.zeros_like(occ_ref)
        occs_ref[...] = jnp.zeros_like(occs_ref)

    raw_ref[...] += raw_p * (1.0 / NUM_PARTS)
    occ_ref[...] += occ_p * (1.0 / NUM_PARTS)
    occs_ref[...] += occ_p * (iota8 == p).astype(jnp.float32)


@jax.jit
def kernel(tpts, viewdir, tflag, dists, part_dist, poses, all_poses, tbounds,
           coord_line, feat_line, W1, b1, W2, b2):
    del part_dist
    N = tpts.shape[0]

    ap3 = all_poses.reshape(POSE_NUM, 24, 3)
    qp3 = poses.reshape(1, 24, 3)
    pid, wscale = pl.pallas_call(
        _pose_select_body,
        out_shape=(
            jax.ShapeDtypeStruct((NUM_PARTS,), jnp.int32),
            jax.ShapeDtypeStruct((NUM_PARTS,), jnp.float32),
        ),
        out_specs=(
            pl.BlockSpec(memory_space=pltpu.SMEM),
            pl.BlockSpec(memory_space=pltpu.SMEM),
        ),
    )(ap3, qp3)

    tshift = jnp.concatenate([coord_line[..., 1:], coord_line[..., -1:]],
                             axis=-1)
    tab = jnp.concatenate([coord_line, tshift], axis=2).astype(jnp.bfloat16)

    tpts24 = tpts.reshape(N, 24)
    view24 = viewdir.reshape(N, 24)
    tflag_f = tflag.astype(jnp.float32)
    dists2 = dists.reshape(N, 1)
    W1p = W1[:, 0:3, :].reshape(24, H)
    W1v = W1[:, 3:6, :].reshape(24, H)
    W1f = W1[:, 6:, :].reshape(NUM_PARTS, 3, PD, H)

    B = 2048
    grid = (N // B, NUM_PARTS)
    out_shapes = (
        jax.ShapeDtypeStruct((N, 4), jnp.float32),
        jax.ShapeDtypeStruct((N, 1), jnp.float32),
        jax.ShapeDtypeStruct((N, NUM_PARTS), jnp.float32),
    )
    grid_spec = pltpu.PrefetchScalarGridSpec(
        num_scalar_prefetch=1,
        grid=grid,
        in_specs=[
            pl.BlockSpec((B, 24), lambda i, p, pid_ref: (i, 0)),
            pl.BlockSpec((B, 24), lambda i, p, pid_ref: (i, 0)),
            pl.BlockSpec((B, NUM_PARTS), lambda i, p, pid_ref: (i, 0)),
            pl.BlockSpec((B, 1), lambda i, p, pid_ref: (i, 0)),
            pl.BlockSpec((1, 2, 3), lambda i, p, pid_ref: (0, 0, 0)),
            pl.BlockSpec((3, 1, 2 * NC, G),
                         lambda i, p, pid_ref: (0, pid_ref[p], 0, 0)),
            pl.BlockSpec((3, 1, NC, PD),
                         lambda i, p, pid_ref: (0, pid_ref[p], 0, 0)),
            pl.BlockSpec((24, H), lambda i, p, pid_ref: (0, 0)),
            pl.BlockSpec((24, H), lambda i, p, pid_ref: (0, 0)),
            pl.BlockSpec((1, 3, PD, H), lambda i, p, pid_ref: (p, 0, 0, 0)),
            pl.BlockSpec((1, 1, H), lambda i, p, pid_ref: (p, 0, 0)),
            pl.BlockSpec((1, H, 5), lambda i, p, pid_ref: (p, 0, 0)),
            pl.BlockSpec((1, 1, 5), lambda i, p, pid_ref: (p, 0, 0)),
            pl.BlockSpec(memory_space=pltpu.SMEM),
        ],
        out_specs=(
            pl.BlockSpec((B, 4), lambda i, p, pid_ref: (i, 0)),
            pl.BlockSpec((B, 1), lambda i, p, pid_ref: (i, 0)),
            pl.BlockSpec((B, NUM_PARTS), lambda i, p, pid_ref: (i, 0)),
        ),
    )
    raw, occ, occs = pl.pallas_call(
        _main_body,
        grid_spec=grid_spec,
        out_shape=out_shapes,
    )(pid, tpts24, view24, tflag_f, dists2, tbounds, tab, feat_line,
      W1p, W1v, W1f, b1.reshape(NUM_PARTS, 1, H), W2,
      b2.reshape(NUM_PARTS, 1, 5), wscale)
    return raw, occ, occs.reshape(N, NUM_PARTS, 1)


# speedup vs baseline: 4.6155x; 1.0532x over previous
import jax
import jax.numpy as jnp
from jax.experimental import pallas as pl
from jax.experimental.pallas import tpu as pltpu

NUM_PARTS = 8
POSE_NUM = 16
NC = 48
G = 512
PD = 20
H = 128
MAT_MODE = ((0, 1), (0, 2), (1, 2))


def _pose_select_body(ap_ref, qp_ref, pid_ref, wscale_ref):
    ap = ap_ref[...]
    qp = qp_ref[...]

    def quat(aa):
        angle = jnp.sqrt(jnp.sum(aa * aa, axis=-1))
        half = 0.5 * angle
        small = angle < 1e-6
        sinc = jnp.where(small, 0.5 - angle * angle / 48.0,
                         jnp.sin(half) / jnp.where(small, 1.0, angle))
        return jnp.cos(half), aa * sinc[..., None]

    kw, kxyz = quat(ap)
    qw, qxyz = quat(qp)
    dot = kw * qw + jnp.sum(kxyz * qxyz, axis=-1)
    absdot = jnp.abs(dot)
    iota = jax.lax.broadcasted_iota(jnp.int32, (POSE_NUM, 1), 0)
    for p in range(NUM_PARTS):
        pd = jnp.sum(absdot[:, 3 * p:3 * p + 3], axis=-1, keepdims=True)
        maxv = jnp.max(pd)
        pid = jnp.min(jnp.where(pd == maxv, iota, POSE_NUM))
        pid_ref[p] = pid.astype(jnp.int32)
        wscale_ref[p] = maxv / jnp.maximum(maxv, 1e-16)


def _main_body(pid_ref, tpts_ref, view_ref, tflag_ref, dists_ref, tb_ref,
               tab_ref, feat_ref, w1p_ref, w1v_ref, w1f_ref, b1_ref, w2_ref,
               b2_ref, wsc_ref, raw_ref, occ_ref, occs_ref):
    p = pl.program_id(1)
    B = dists_ref.shape[0]
    dists = dists_ref[...]
    lane_iota = jax.lax.broadcasted_iota(jnp.int16, (B, G), 1)
    lane96 = jax.lax.broadcasted_iota(jnp.int32, (B, 2 * NC), 1)
    lane24 = jax.lax.broadcasted_iota(jnp.int32, (B, 24), 1)
    iota8 = jax.lax.broadcasted_iota(jnp.int32, (B, NUM_PARTS), 1)
    sub24 = jax.lax.broadcasted_iota(jnp.int32, (24, 1), 0)
    r96 = jax.lax.broadcasted_iota(jnp.int32, (2 * NC, NC), 0)
    c48 = jax.lax.broadcasted_iota(jnp.int32, (2 * NC, NC), 1)
    fold = jnp.logical_or(r96 == c48, r96 == c48 + NC).astype(jnp.bfloat16)

    tpts24 = tpts_ref[...]
    view24 = view_ref[...]
    lines = []
    for ax in range(3):
        sel = (lane24 == 3 * p + ax).astype(jnp.float32)
        x = jnp.sum(tpts24 * sel, axis=1, keepdims=True)
        xmin = tb_ref[0, 0, ax]
        xmax = tb_ref[0, 1, ax]
        gax = jnp.clip((x - xmin) * ((G - 1.0) / (xmax - xmin)), 0.0, G - 1.0)
        g0f = jnp.floor(gax)
        tax = gax - g0f
        g0 = g0f.astype(jnp.int16)
        onehot = (lane_iota == g0).astype(jnp.bfloat16)
        vv = jax.lax.dot_general(
            onehot, tab_ref[ax, 0], (((1,), (1,)), ((), ())),
            preferred_element_type=jnp.float32)
        ww = jnp.where(lane96 < NC, 1.0 - tax, tax)
        uu = (vv * ww).astype(jnp.bfloat16)
        lines.append(jnp.dot(uu, fold, preferred_element_type=jnp.float32))

    pmask = jnp.logical_and(sub24 >= 3 * p, sub24 < 3 * p + 3)
    pm = pmask.astype(jnp.float32)
    h_pre = (jnp.dot(tpts24, w1p_ref[...] * pm,
                     preferred_element_type=jnp.float32)
             + jnp.dot(view24, w1v_ref[...] * pm,
                       preferred_element_type=jnp.float32))
    wsc = wsc_ref[p]
    for ip, (i0, i1) in enumerate(MAT_MODE):
        prod = lines[i0] * lines[i1]
        fw = jnp.dot(feat_ref[ip, 0], w1f_ref[0, ip],
                     preferred_element_type=jnp.float32) * wsc
        h_pre = h_pre + jnp.dot(prod, fw, preferred_element_type=jnp.float32)
    h = jnp.maximum(h_pre + b1_ref[0, 0, :], 0.0)
    out = jnp.dot(h, w2_ref[0], preferred_element_type=jnp.float32) + b2_ref[0, 0, :]
    m = jnp.sum(tflag_ref[...] * (iota8 == p).astype(jnp.float32),
                axis=1, keepdims=True)
    raw_p = out[:, :4] * m
    occ_p = (1.0 - jnp.exp(-jnp.maximum(out[:, 4:5], 0.0) * dists)) * m

    @pl.when(p == 0)
    def _():
        raw_ref[...] = jnp.zeros_like(raw_ref)
        occ_ref[...] = jnp.zeros_like(occ_ref)
        occs_ref[...] = jnp.zeros_like(occs_ref)

    raw_ref[...] += raw_p * (1.0 / NUM_PARTS)
    occ_ref[...] += occ_p * (1.0 / NUM_PARTS)
    occs_ref[...] += occ_p * (iota8 == p).astype(jnp.float32)


@jax.jit
def kernel(tpts, viewdir, tflag, dists, part_dist, poses, all_poses, tbounds,
           coord_line, feat_line, W1, b1, W2, b2):
    del part_dist
    N = tpts.shape[0]

    ap3 = all_poses.reshape(POSE_NUM, 24, 3)
    qp3 = poses.reshape(1, 24, 3)
    pid, wscale = pl.pallas_call(
        _pose_select_body,
        out_shape=(
            jax.ShapeDtypeStruct((NUM_PARTS,), jnp.int32),
            jax.ShapeDtypeStruct((NUM_PARTS,), jnp.float32),
        ),
        out_specs=(
            pl.BlockSpec(memory_space=pltpu.SMEM),
            pl.BlockSpec(memory_space=pltpu.SMEM),
        ),
    )(ap3, qp3)

    tshift = jnp.concatenate([coord_line[..., 1:], coord_line[..., -1:]],
                             axis=-1)
    tab = jnp.concatenate([coord_line, tshift], axis=2).astype(jnp.bfloat16)

    tpts24 = tpts.reshape(N, 24)
    view24 = viewdir.reshape(N, 24)
    tflag_f = tflag.astype(jnp.float32)
    dists2 = dists.reshape(N, 1)
    W1p = W1[:, 0:3, :].reshape(24, H)
    W1v = W1[:, 3:6, :].reshape(24, H)
    W1f = W1[:, 6:, :].reshape(NUM_PARTS, 3, PD, H)

    B = 4096
    grid = (N // B, NUM_PARTS)
    out_shapes = (
        jax.ShapeDtypeStruct((N, 4), jnp.float32),
        jax.ShapeDtypeStruct((N, 1), jnp.float32),
        jax.ShapeDtypeStruct((N, NUM_PARTS), jnp.float32),
    )
    grid_spec = pltpu.PrefetchScalarGridSpec(
        num_scalar_prefetch=1,
        grid=grid,
        in_specs=[
            pl.BlockSpec((B, 24), lambda i, p, pid_ref: (i, 0)),
            pl.BlockSpec((B, 24), lambda i, p, pid_ref: (i, 0)),
            pl.BlockSpec((B, NUM_PARTS), lambda i, p, pid_ref: (i, 0)),
            pl.BlockSpec((B, 1), lambda i, p, pid_ref: (i, 0)),
            pl.BlockSpec((1, 2, 3), lambda i, p, pid_ref: (0, 0, 0)),
            pl.BlockSpec((3, 1, 2 * NC, G),
                         lambda i, p, pid_ref: (0, pid_ref[p], 0, 0)),
            pl.BlockSpec((3, 1, NC, PD),
                         lambda i, p, pid_ref: (0, pid_ref[p], 0, 0)),
            pl.BlockSpec((24, H), lambda i, p, pid_ref: (0, 0)),
            pl.BlockSpec((24, H), lambda i, p, pid_ref: (0, 0)),
            pl.BlockSpec((1, 3, PD, H), lambda i, p, pid_ref: (p, 0, 0, 0)),
            pl.BlockSpec((1, 1, H), lambda i, p, pid_ref: (p, 0, 0)),
            pl.BlockSpec((1, H, 5), lambda i, p, pid_ref: (p, 0, 0)),
            pl.BlockSpec((1, 1, 5), lambda i, p, pid_ref: (p, 0, 0)),
            pl.BlockSpec(memory_space=pltpu.SMEM),
        ],
        out_specs=(
            pl.BlockSpec((B, 4), lambda i, p, pid_ref: (i, 0)),
            pl.BlockSpec((B, 1), lambda i, p, pid_ref: (i, 0)),
            pl.BlockSpec((B, NUM_PARTS), lambda i, p, pid_ref: (i, 0)),
        ),
    )
    raw, occ, occs = pl.pallas_call(
        _main_body,
        grid_spec=grid_spec,
        out_shape=out_shapes,
    )(pid, tpts24, view24, tflag_f, dists2, tbounds, tab, feat_line,
      W1p, W1v, W1f, b1.reshape(NUM_PARTS, 1, H), W2,
      b2.reshape(NUM_PARTS, 1, 5), wscale)
    return raw, occ, occs.reshape(N, NUM_PARTS, 1)
